# baseline (speedup 1.0000x reference)
import functools

import jax
import jax.numpy as jnp
from jax import lax
from jax.experimental import pallas as pl
from jax.experimental.pallas import tpu as pltpu
from jax.experimental.pallas import tpu_sc as plsc

NC = 2
NS = 16
LN = 16
WIN = 128
K = 12
ROWB = 1024


def _mesh():
    return plsc.VectorSubcoreMesh(core_axis_name="c", subcore_axis_name="s")


_SC_PARAMS = pltpu.CompilerParams(use_tc_tiling_on_sc=False)


def _sc_hist(dst2d, ones_w, zeros_stripe, nt):
    wins_per = dst2d.shape[0] // (NC * NS)
    groups = wins_per // K
    stripe = nt // NS

    @functools.partial(
        pl.kernel,
        out_type=jax.ShapeDtypeStruct((NC * nt, LN), jnp.float32),
        mesh=_mesh(),
        compiler_params=_SC_PARAMS,
        scratch_types=[
            pltpu.VMEM((2, K, WIN), jnp.int32),
            pltpu.VMEM((WIN, LN), jnp.float32),
            pltpu.VMEM_SHARED((nt, LN), jnp.float32),
            pltpu.SemaphoreType.DMA,
            pltpu.SemaphoreType.DMA((2,)),
        ],
    )
    def k(dst_hbm, ones_hbm, zeros_hbm, out_hbm, di_v, val_v, acc, isem, ssems):
        c = lax.axis_index("c")
        s = lax.axis_index("s")
        pltpu.sync_copy(zeros_hbm, acc.at[pl.ds(s * stripe, stripe)])
        pltpu.sync_copy(ones_hbm, val_v)
        plsc.subcore_barrier()
        base = (c * NS + s) * wins_per

        pltpu.async_copy(dst_hbm.at[pl.ds(base, K)], di_v.at[0], isem)

        @pl.loop(0, groups)
        def _(g):
            b = g & 1
            pltpu.make_async_copy(
                dst_hbm.at[pl.ds(base, K)], di_v.at[b], isem
            ).wait()

            @pl.when(g > 0)
            def _():
                for j in range(K):
                    pltpu.make_async_copy(
                        val_v, acc.at[di_v.at[1 - b].at[j]], ssems.at[1 - b]
                    ).wait()

            @pl.when(g + 1 < groups)
            def _():
                pltpu.async_copy(
                    dst_hbm.at[pl.ds(base + (g + 1) * K, K)],
                    di_v.at[1 - b], isem,
                )

            for j in range(K):
                pltpu.async_copy(
                    val_v, acc.at[di_v.at[b].at[j]], ssems.at[b], add=True
                )

        bl = (groups - 1) & 1
        for j in range(K):
            pltpu.make_async_copy(
                val_v, acc.at[di_v.at[bl].at[j]], ssems.at[bl]
            ).wait()
        plsc.subcore_barrier()
        pltpu.sync_copy(
            acc.at[pl.ds(s * stripe, stripe)],
            out_hbm.at[pl.ds(c * nt + s * stripe, stripe)],
        )

    return k(dst2d, ones_w, zeros_stripe)


def _edge_loop(src_hbm, dst_hbm, tab, acc, si_v, di_v, rows_v,
               isem, gsems, ssem, base, wins_per):
    groups = wins_per // K

    pltpu.async_copy(src_hbm.at[pl.ds(base, K)], si_v.at[0], isem)
    pltpu.async_copy(dst_hbm.at[pl.ds(base, K)], di_v.at[0], isem)

    @pl.loop(0, groups)
    def _(g):
        b = g & 1
        pltpu.make_async_copy(
            src_hbm.at[pl.ds(base, K)], si_v.at[b], isem
        ).wait()
        pltpu.make_async_copy(
            dst_hbm.at[pl.ds(base, K)], di_v.at[b], isem
        ).wait()

        @pl.when(g > 0)
        def _():
            for j in range(K):
                pltpu.make_async_copy(
                    rows_v.at[j], acc.at[di_v.at[1 - b].at[j]], ssem
                ).wait()

        @pl.when(g + 1 < groups)
        def _():
            pltpu.async_copy(
                src_hbm.at[pl.ds(base + (g + 1) * K, K)], si_v.at[1 - b], isem
            )
            pltpu.async_copy(
                dst_hbm.at[pl.ds(base + (g + 1) * K, K)], di_v.at[1 - b], isem
            )

        for j in range(K):
            pltpu.async_copy(
                tab.at[si_v.at[b].at[j]], rows_v.at[j], gsems.at[j]
            )
        for j in range(K):
            pltpu.make_async_copy(
                tab.at[si_v.at[b].at[j]], rows_v.at[j], gsems.at[j]
            ).wait()
            pltpu.async_copy(
                rows_v.at[j], acc.at[di_v.at[b].at[j]], ssem, add=True,
            )

    bl = (groups - 1) & 1
    for j in range(K):
        pltpu.make_async_copy(
            rows_v.at[j], acc.at[di_v.at[bl].at[j]], ssem
        ).wait()


def _sc_edge_half(src2d, dst2d, tab0, tab1, zeros_stripe, nt):
    wins_per = src2d.shape[0] // NS
    stripe = nt // NS

    @functools.partial(
        pl.kernel,
        out_type=jax.ShapeDtypeStruct((NC * nt, LN), jnp.float32),
        mesh=_mesh(),
        compiler_params=_SC_PARAMS,
        scratch_types=[
            pltpu.VMEM((2, K, WIN), jnp.int32),
            pltpu.VMEM((2, K, WIN), jnp.int32),
            pltpu.VMEM((K, WIN, LN), jnp.float32),
            pltpu.VMEM_SHARED((nt, LN), jnp.float32),
            pltpu.SemaphoreType.DMA,
            pltpu.SemaphoreType.DMA((K,)),
            pltpu.SemaphoreType.DMA,
        ],
    )
    def k(src_hbm, dst_hbm, t0_hbm, t1_hbm, zeros_hbm, out_hbm,
          si_v, di_v, rows_v, acc, isem, gsems, ssem):
        c = lax.axis_index("c")
        s = lax.axis_index("s")
        pltpu.sync_copy(zeros_hbm, acc.at[pl.ds(s * stripe, stripe)])
        plsc.subcore_barrier()
        base = s * wins_per

        @pl.when(c == 0)
        def _():
            _edge_loop(src_hbm, dst_hbm, t0_hbm, acc, si_v, di_v, rows_v,
                       isem, gsems, ssem, base, wins_per)

        @pl.when(c == 1)
        def _():
            _edge_loop(src_hbm, dst_hbm, t1_hbm, acc, si_v, di_v, rows_v,
                       isem, gsems, ssem, base, wins_per)

        plsc.subcore_barrier()
        pltpu.sync_copy(
            acc.at[pl.ds(s * stripe, stripe)],
            out_hbm.at[pl.ds(c * nt + s * stripe, stripe)],
        )

    return k(src2d, dst2d, tab0, tab1, zeros_stripe)


def _sc_edge_split(src2d, dst2d, tab, zeros_stripe, nt):
    wins_per = src2d.shape[0] // (NC * NS)
    stripe = nt // NS

    @functools.partial(
        pl.kernel,
        out_type=jax.ShapeDtypeStruct((NC * nt, LN), jnp.float32),
        mesh=_mesh(),
        compiler_params=_SC_PARAMS,
        scratch_types=[
            pltpu.VMEM((2, K, WIN), jnp.int32),
            pltpu.VMEM((2, K, WIN), jnp.int32),
            pltpu.VMEM((K, WIN, LN), jnp.float32),
            pltpu.VMEM_SHARED((nt, LN), jnp.float32),
            pltpu.SemaphoreType.DMA,
            pltpu.SemaphoreType.DMA((K,)),
            pltpu.SemaphoreType.DMA,
        ],
    )
    def k(src_hbm, dst_hbm, tab_hbm, zeros_hbm, out_hbm,
          si_v, di_v, rows_v, acc, isem, gsems, ssem):
        c = lax.axis_index("c")
        s = lax.axis_index("s")
        pltpu.sync_copy(zeros_hbm, acc.at[pl.ds(s * stripe, stripe)])
        plsc.subcore_barrier()
        base = (c * NS + s) * wins_per
        _edge_loop(src_hbm, dst_hbm, tab_hbm, acc, si_v, di_v, rows_v,
                   isem, gsems, ssem, base, wins_per)
        plsc.subcore_barrier()
        pltpu.sync_copy(
            acc.at[pl.ds(s * stripe, stripe)],
            out_hbm.at[pl.ds(c * nt + s * stripe, stripe)],
        )

    return k(src2d, dst2d, tab, zeros_stripe)


def _tc_prep(xpad, W1, degp, nt):
    grid = nt // ROWB
    d_in, d_out = W1.shape

    def body(x_ref, w_ref, dg_ref, dinv_ref, a_ref, b_ref):
        deg = dg_ref[0][:, 0:1] + dg_ref[1][:, 0:1] + 1.0
        dinv = lax.rsqrt(deg)
        h = jnp.dot(x_ref[...], w_ref[...], preferred_element_type=jnp.float32)
        hp = h * dinv
        dinv_ref[...] = dinv
        a_ref[...] = hp[:, : d_out // 2]
        b_ref[...] = hp[:, d_out // 2:]

    return pl.pallas_call(
        body,
        grid=(grid,),
        in_specs=[
            pl.BlockSpec((ROWB, d_in), lambda i: (i, 0)),
            pl.BlockSpec((d_in, d_out), lambda i: (0, 0)),
            pl.BlockSpec((2, ROWB, LN), lambda i: (0, i, 0)),
        ],
        out_specs=[
            pl.BlockSpec((ROWB, 1), lambda i: (i, 0)),
            pl.BlockSpec((ROWB, d_out // 2), lambda i: (i, 0)),
            pl.BlockSpec((ROWB, d_out // 2), lambda i: (i, 0)),
        ],
        out_shape=[
            jax.ShapeDtypeStruct((nt, 1), jnp.float32),
            jax.ShapeDtypeStruct((nt, d_out // 2), jnp.float32),
            jax.ShapeDtypeStruct((nt, d_out // 2), jnp.float32),
        ],
    )(xpad, W1, degp)


def _tc_mid(agg1, h1a, h1b, dinv, b1, W2, nt):
    grid = nt // ROWB
    d_hid, d_out = W2.shape

    def body(agg_ref, a_ref, b_ref, dinv_ref, b1_ref, w2_ref, out_ref):
        dv = dinv_ref[...]
        xa = (agg_ref[0] + a_ref[...]) * dv
        xb = (agg_ref[1] + b_ref[...]) * dv
        x2 = jnp.concatenate([xa, xb], axis=1) + b1_ref[...]
        x2 = jnp.maximum(x2, 0.0)
        h2 = jnp.dot(x2, w2_ref[...], preferred_element_type=jnp.float32)
        out_ref[...] = h2 * dv

    return pl.pallas_call(
        body,
        grid=(grid,),
        in_specs=[
            pl.BlockSpec((2, ROWB, LN), lambda i: (0, i, 0)),
            pl.BlockSpec((ROWB, LN), lambda i: (i, 0)),
            pl.BlockSpec((ROWB, LN), lambda i: (i, 0)),
            pl.BlockSpec((ROWB, 1), lambda i: (i, 0)),
            pl.BlockSpec((1, d_hid), lambda i: (0, 0)),
            pl.BlockSpec((d_hid, d_out), lambda i: (0, 0)),
        ],
        out_specs=pl.BlockSpec((ROWB, d_out), lambda i: (i, 0)),
        out_shape=jax.ShapeDtypeStruct((nt, d_out), jnp.float32),
    )(agg1, h1a, h1b, dinv, b1, W2)


def _tc_final(agg2, h2p, dinv, b2, nt):
    grid = nt // ROWB

    def body(agg_ref, h_ref, dinv_ref, b2_ref, out_ref):
        out_ref[...] = (
            (agg_ref[0] + agg_ref[1] + h_ref[...]) * dinv_ref[...] + b2_ref[...]
        )

    return pl.pallas_call(
        body,
        grid=(grid,),
        in_specs=[
            pl.BlockSpec((2, ROWB, LN), lambda i: (0, i, 0)),
            pl.BlockSpec((ROWB, LN), lambda i: (i, 0)),
            pl.BlockSpec((ROWB, 1), lambda i: (i, 0)),
            pl.BlockSpec((1, LN), lambda i: (0, 0)),
        ],
        out_specs=pl.BlockSpec((ROWB, LN), lambda i: (i, 0)),
        out_shape=jax.ShapeDtypeStruct((nt, LN), jnp.float32),
    )(agg2, h2p, dinv, b2)


def kernel(edge_index, user_emb, item_emb, W1, b1, W2, b2):
    nu = user_emb.shape[0]
    nn = nu + item_emb.shape[0]
    ne = edge_index.shape[1]
    nt = ((nn + ROWB) // ROWB) * ROWB
    nsc = ((nn + NS) // NS) * NS
    chunk = NC * NS * WIN * K
    nep = -(-ne // chunk) * chunk

    src = jnp.pad(edge_index[0], (0, nep - ne), constant_values=nn)
    dst = jnp.pad(edge_index[1], (0, nep - ne), constant_values=nn)
    src2d = src.reshape(nep // WIN, WIN)
    dst2d = dst.reshape(nep // WIN, WIN)

    x = jnp.concatenate([user_emb, item_emb], axis=0)
    xpad = jnp.pad(x, ((0, nt - nn), (0, 0)))
    zeros_stripe = jnp.zeros((nsc // NS, LN), jnp.float32)
    ones_w = jnp.ones((WIN, LN), jnp.float32)

    def sc_pad(p):
        p = p.reshape(NC, nsc, LN)
        return jnp.pad(p, ((0, 0), (0, nt - nsc), (0, 0)))

    degp = sc_pad(_sc_hist(dst2d, ones_w, zeros_stripe, nsc))
    dinv, h1a, h1b = _tc_prep(xpad, W1, degp, nt)
    agg1 = sc_pad(_sc_edge_half(src2d, dst2d, h1a, h1b, zeros_stripe, nsc))
    h2p = _tc_mid(agg1, h1a, h1b, dinv, b1.reshape(1, -1), W2, nt)
    agg2 = sc_pad(_sc_edge_split(src2d, dst2d, h2p, zeros_stripe, nsc))
    out = _tc_final(agg2, h2p, dinv, b2.reshape(1, -1), nt)

    return out[:nu], out[nu:nn]


# speedup vs baseline: 1.0967x; 1.0967x over previous
import functools

import jax
import jax.numpy as jnp
from jax import lax
from jax.experimental import pallas as pl
from jax.experimental.pallas import tpu as pltpu
from jax.experimental.pallas import tpu_sc as plsc

NC = 2
NS = 16
LN = 16
WIN = 128
K = 6
ROWB = 1024


def _mesh():
    return plsc.VectorSubcoreMesh(core_axis_name="c", subcore_axis_name="s")


_SC_PARAMS = pltpu.CompilerParams(use_tc_tiling_on_sc=False)


def _sc_hist(dst2d, ones_w, zeros_stripe, nt):
    wins_per = dst2d.shape[0] // (NC * NS)
    groups = wins_per // K
    stripe = nt // NS

    @functools.partial(
        pl.kernel,
        out_type=jax.ShapeDtypeStruct((NC * nt, LN), jnp.float32),
        mesh=_mesh(),
        compiler_params=_SC_PARAMS,
        scratch_types=[
            pltpu.VMEM((2, K, WIN), jnp.int32),
            pltpu.VMEM((WIN, LN), jnp.float32),
            pltpu.VMEM_SHARED((nt, LN), jnp.float32),
            pltpu.SemaphoreType.DMA,
            pltpu.SemaphoreType.DMA((2,)),
        ],
    )
    def k(dst_hbm, ones_hbm, zeros_hbm, out_hbm, di_v, val_v, acc, isem, ssems):
        c = lax.axis_index("c")
        s = lax.axis_index("s")
        pltpu.sync_copy(zeros_hbm, acc.at[pl.ds(s * stripe, stripe)])
        pltpu.sync_copy(ones_hbm, val_v)
        plsc.subcore_barrier()
        base = (c * NS + s) * wins_per

        pltpu.async_copy(dst_hbm.at[pl.ds(base, K)], di_v.at[0], isem)

        @pl.loop(0, groups)
        def _(g):
            b = g & 1
            pltpu.make_async_copy(
                dst_hbm.at[pl.ds(base, K)], di_v.at[b], isem
            ).wait()

            for j in range(K):
                pltpu.async_copy(
                    val_v, acc.at[di_v.at[b].at[j]], ssems.at[b], add=True
                )

            @pl.when(g > 0)
            def _():
                for j in range(K):
                    pltpu.make_async_copy(
                        val_v, acc.at[di_v.at[1 - b].at[j]], ssems.at[1 - b]
                    ).wait()

            @pl.when(g + 1 < groups)
            def _():
                pltpu.async_copy(
                    dst_hbm.at[pl.ds(base + (g + 1) * K, K)],
                    di_v.at[1 - b], isem,
                )

        bl = (groups - 1) & 1
        for j in range(K):
            pltpu.make_async_copy(
                val_v, acc.at[di_v.at[bl].at[j]], ssems.at[bl]
            ).wait()
        plsc.subcore_barrier()
        pltpu.sync_copy(
            acc.at[pl.ds(s * stripe, stripe)],
            out_hbm.at[pl.ds(c * nt + s * stripe, stripe)],
        )

    return k(dst2d, ones_w, zeros_stripe)


def _edge_loop(src_hbm, dst_hbm, tab, acc, si_v, di_v, rows_v,
               isem, gsems, ssems, base, wins_per):
    groups = wins_per // K

    pltpu.async_copy(src_hbm.at[pl.ds(base, K)], si_v.at[0], isem)
    pltpu.async_copy(dst_hbm.at[pl.ds(base, K)], di_v.at[0], isem)

    @pl.loop(0, groups)
    def _(g):
        b = g & 1
        pltpu.make_async_copy(
            src_hbm.at[pl.ds(base, K)], si_v.at[b], isem
        ).wait()
        pltpu.make_async_copy(
            dst_hbm.at[pl.ds(base, K)], di_v.at[b], isem
        ).wait()

        for j in range(K):
            pltpu.async_copy(
                tab.at[si_v.at[b].at[j]], rows_v.at[b].at[j], gsems.at[j]
            )

        @pl.when(g > 0)
        def _():
            for j in range(K):
                pltpu.make_async_copy(
                    rows_v.at[1 - b].at[j],
                    acc.at[di_v.at[1 - b].at[j]],
                    ssems.at[1 - b],
                ).wait()

        @pl.when(g + 1 < groups)
        def _():
            pltpu.async_copy(
                src_hbm.at[pl.ds(base + (g + 1) * K, K)], si_v.at[1 - b], isem
            )
            pltpu.async_copy(
                dst_hbm.at[pl.ds(base + (g + 1) * K, K)], di_v.at[1 - b], isem
            )
        for j in range(K):
            pltpu.make_async_copy(
                tab.at[si_v.at[b].at[j]], rows_v.at[b].at[j], gsems.at[j]
            ).wait()
            pltpu.async_copy(
                rows_v.at[b].at[j], acc.at[di_v.at[b].at[j]],
                ssems.at[b], add=True,
            )

    bl = (groups - 1) & 1
    for j in range(K):
        pltpu.make_async_copy(
            rows_v.at[bl].at[j], acc.at[di_v.at[bl].at[j]], ssems.at[bl]
        ).wait()


def _sc_edge_half(src2d, dst2d, tab0, tab1, zeros_stripe, nt):
    wins_per = src2d.shape[0] // NS
    stripe = nt // NS

    @functools.partial(
        pl.kernel,
        out_type=jax.ShapeDtypeStruct((NC * nt, LN), jnp.float32),
        mesh=_mesh(),
        compiler_params=_SC_PARAMS,
        scratch_types=[
            pltpu.VMEM((2, K, WIN), jnp.int32),
            pltpu.VMEM((2, K, WIN), jnp.int32),
            pltpu.VMEM((2, K, WIN, LN), jnp.float32),
            pltpu.VMEM_SHARED((nt, LN), jnp.float32),
            pltpu.SemaphoreType.DMA,
            pltpu.SemaphoreType.DMA((K,)),
            pltpu.SemaphoreType.DMA((2,)),
        ],
    )
    def k(src_hbm, dst_hbm, t0_hbm, t1_hbm, zeros_hbm, out_hbm,
          si_v, di_v, rows_v, acc, isem, gsems, ssems):
        c = lax.axis_index("c")
        s = lax.axis_index("s")
        pltpu.sync_copy(zeros_hbm, acc.at[pl.ds(s * stripe, stripe)])
        plsc.subcore_barrier()
        base = s * wins_per

        @pl.when(c == 0)
        def _():
            _edge_loop(src_hbm, dst_hbm, t0_hbm, acc, si_v, di_v, rows_v,
                       isem, gsems, ssems, base, wins_per)

        @pl.when(c == 1)
        def _():
            _edge_loop(src_hbm, dst_hbm, t1_hbm, acc, si_v, di_v, rows_v,
                       isem, gsems, ssems, base, wins_per)

        plsc.subcore_barrier()
        pltpu.sync_copy(
            acc.at[pl.ds(s * stripe, stripe)],
            out_hbm.at[pl.ds(c * nt + s * stripe, stripe)],
        )

    return k(src2d, dst2d, tab0, tab1, zeros_stripe)


def _sc_edge_split(src2d, dst2d, tab, zeros_stripe, nt):
    wins_per = src2d.shape[0] // (NC * NS)
    stripe = nt // NS

    @functools.partial(
        pl.kernel,
        out_type=jax.ShapeDtypeStruct((NC * nt, LN), jnp.float32),
        mesh=_mesh(),
        compiler_params=_SC_PARAMS,
        scratch_types=[
            pltpu.VMEM((2, K, WIN), jnp.int32),
            pltpu.VMEM((2, K, WIN), jnp.int32),
            pltpu.VMEM((2, K, WIN, LN), jnp.float32),
            pltpu.VMEM_SHARED((nt, LN), jnp.float32),
            pltpu.SemaphoreType.DMA,
            pltpu.SemaphoreType.DMA((K,)),
            pltpu.SemaphoreType.DMA((2,)),
        ],
    )
    def k(src_hbm, dst_hbm, tab_hbm, zeros_hbm, out_hbm,
          si_v, di_v, rows_v, acc, isem, gsems, ssems):
        c = lax.axis_index("c")
        s = lax.axis_index("s")
        pltpu.sync_copy(zeros_hbm, acc.at[pl.ds(s * stripe, stripe)])
        plsc.subcore_barrier()
        base = (c * NS + s) * wins_per
        _edge_loop(src_hbm, dst_hbm, tab_hbm, acc, si_v, di_v, rows_v,
                   isem, gsems, ssems, base, wins_per)
        plsc.subcore_barrier()
        pltpu.sync_copy(
            acc.at[pl.ds(s * stripe, stripe)],
            out_hbm.at[pl.ds(c * nt + s * stripe, stripe)],
        )

    return k(src2d, dst2d, tab, zeros_stripe)


def _tc_mm(xpad, W1, nt):
    grid = nt // ROWB
    d_in, d_out = W1.shape

    def body(x_ref, w_ref, h_ref):
        h_ref[...] = jnp.dot(
            x_ref[...], w_ref[...], preferred_element_type=jnp.float32
        )

    return pl.pallas_call(
        body,
        grid=(grid,),
        in_specs=[
            pl.BlockSpec((ROWB, d_in), lambda i: (i, 0)),
            pl.BlockSpec((d_in, d_out), lambda i: (0, 0)),
        ],
        out_specs=pl.BlockSpec((ROWB, d_out), lambda i: (i, 0)),
        out_shape=jax.ShapeDtypeStruct((nt, d_out), jnp.float32),
    )(xpad, W1)


def _tc_prep(h1, degp, nt):
    grid = nt // ROWB
    d_out = h1.shape[1]

    def body(h_ref, dg_ref, dinv_ref, a_ref, b_ref):
        deg = dg_ref[0][:, 0:1] + dg_ref[1][:, 0:1] + 1.0
        dinv = lax.rsqrt(deg)
        hp = h_ref[...] * dinv
        dinv_ref[...] = dinv
        a_ref[...] = hp[:, : d_out // 2]
        b_ref[...] = hp[:, d_out // 2:]

    return pl.pallas_call(
        body,
        grid=(grid,),
        in_specs=[
            pl.BlockSpec((ROWB, d_out), lambda i: (i, 0)),
            pl.BlockSpec((2, ROWB, LN), lambda i: (0, i, 0)),
        ],
        out_specs=[
            pl.BlockSpec((ROWB, 1), lambda i: (i, 0)),
            pl.BlockSpec((ROWB, d_out // 2), lambda i: (i, 0)),
            pl.BlockSpec((ROWB, d_out // 2), lambda i: (i, 0)),
        ],
        out_shape=[
            jax.ShapeDtypeStruct((nt, 1), jnp.float32),
            jax.ShapeDtypeStruct((nt, d_out // 2), jnp.float32),
            jax.ShapeDtypeStruct((nt, d_out // 2), jnp.float32),
        ],
    )(h1, degp)


def _tc_mid(agg1, h1a, h1b, dinv, b1, W2, nt):
    grid = nt // ROWB
    d_hid, d_out = W2.shape

    def body(agg_ref, a_ref, b_ref, dinv_ref, b1_ref, w2_ref, out_ref):
        dv = dinv_ref[...]
        xa = (agg_ref[0] + a_ref[...]) * dv
        xb = (agg_ref[1] + b_ref[...]) * dv
        x2 = jnp.concatenate([xa, xb], axis=1) + b1_ref[...]
        x2 = jnp.maximum(x2, 0.0)
        h2 = jnp.dot(x2, w2_ref[...], preferred_element_type=jnp.float32)
        out_ref[...] = h2 * dv

    return pl.pallas_call(
        body,
        grid=(grid,),
        in_specs=[
            pl.BlockSpec((2, ROWB, LN), lambda i: (0, i, 0)),
            pl.BlockSpec((ROWB, LN), lambda i: (i, 0)),
            pl.BlockSpec((ROWB, LN), lambda i: (i, 0)),
            pl.BlockSpec((ROWB, 1), lambda i: (i, 0)),
            pl.BlockSpec((1, d_hid), lambda i: (0, 0)),
            pl.BlockSpec((d_hid, d_out), lambda i: (0, 0)),
        ],
        out_specs=pl.BlockSpec((ROWB, d_out), lambda i: (i, 0)),
        out_shape=jax.ShapeDtypeStruct((nt, d_out), jnp.float32),
    )(agg1, h1a, h1b, dinv, b1, W2)


def _tc_final(agg2, h2p, dinv, b2, nt):
    grid = nt // ROWB

    def body(agg_ref, h_ref, dinv_ref, b2_ref, out_ref):
        out_ref[...] = (
            (agg_ref[0] + agg_ref[1] + h_ref[...]) * dinv_ref[...] + b2_ref[...]
        )

    return pl.pallas_call(
        body,
        grid=(grid,),
        in_specs=[
            pl.BlockSpec((2, ROWB, LN), lambda i: (0, i, 0)),
            pl.BlockSpec((ROWB, LN), lambda i: (i, 0)),
            pl.BlockSpec((ROWB, 1), lambda i: (i, 0)),
            pl.BlockSpec((1, LN), lambda i: (0, 0)),
        ],
        out_specs=pl.BlockSpec((ROWB, LN), lambda i: (i, 0)),
        out_shape=jax.ShapeDtypeStruct((nt, LN), jnp.float32),
    )(agg2, h2p, dinv, b2)


def kernel(edge_index, user_emb, item_emb, W1, b1, W2, b2):
    nu = user_emb.shape[0]
    nn = nu + item_emb.shape[0]
    ne = edge_index.shape[1]
    nt = ((nn + ROWB) // ROWB) * ROWB
    chunk = NC * NS * WIN * K
    nep = -(-ne // chunk) * chunk

    src = jnp.pad(edge_index[0], (0, nep - ne), constant_values=nn)
    dst = jnp.pad(edge_index[1], (0, nep - ne), constant_values=nn)
    src2d = src.reshape(nep // WIN, WIN)
    dst2d = dst.reshape(nep // WIN, WIN)

    x = jnp.concatenate([user_emb, item_emb], axis=0)
    xpad = jnp.pad(x, ((0, nt - nn), (0, 0)))
    zeros_stripe = jnp.zeros((nt // NS, LN), jnp.float32)
    ones_w = jnp.ones((WIN, LN), jnp.float32)

    h1 = _tc_mm(xpad, W1, nt)
    degp = _sc_hist(dst2d, ones_w, zeros_stripe, nt).reshape(NC, nt, LN)
    dinv, h1a, h1b = _tc_prep(h1, degp, nt)
    agg1 = _sc_edge_half(src2d, dst2d, h1a, h1b, zeros_stripe, nt).reshape(NC, nt, LN)
    h2p = _tc_mid(agg1, h1a, h1b, dinv, b1.reshape(1, -1), W2, nt)
    agg2 = _sc_edge_split(src2d, dst2d, h2p, zeros_stripe, nt).reshape(NC, nt, LN)
    out = _tc_final(agg2, h2p, dinv, b2.reshape(1, -1), nt)

    return out[:nu], out[nu:nn]


# speedup vs baseline: 1.1335x; 1.0336x over previous
import functools

import jax
import jax.numpy as jnp
from jax import lax
from jax.experimental import pallas as pl
from jax.experimental.pallas import tpu as pltpu
from jax.experimental.pallas import tpu_sc as plsc

NC = 2
NS = 16
LN = 16
WIN = 128
K = 6
ROWB = 1024


def _mesh():
    return plsc.VectorSubcoreMesh(core_axis_name="c", subcore_axis_name="s")


_SC_PARAMS = pltpu.CompilerParams(use_tc_tiling_on_sc=False)


def _sc_hist(dst2d, ones_w, zeros_stripe, nt):
    wins_per = dst2d.shape[0] // (NC * NS)
    groups = wins_per // K
    stripe = nt // NS

    @functools.partial(
        pl.kernel,
        out_type=jax.ShapeDtypeStruct((NC * nt, LN), jnp.float32),
        mesh=_mesh(),
        compiler_params=_SC_PARAMS,
        scratch_types=[
            pltpu.VMEM((2, K, WIN), jnp.int32),
            pltpu.VMEM((WIN, LN), jnp.float32),
            pltpu.VMEM_SHARED((nt, LN), jnp.float32),
            pltpu.SemaphoreType.DMA,
            pltpu.SemaphoreType.DMA((2,)),
        ],
    )
    def k(dst_hbm, ones_hbm, zeros_hbm, out_hbm, di_v, val_v, acc, isem, ssems):
        c = lax.axis_index("c")
        s = lax.axis_index("s")
        pltpu.sync_copy(zeros_hbm, acc.at[pl.ds(s * stripe, stripe)])
        pltpu.sync_copy(ones_hbm, val_v)
        plsc.subcore_barrier()
        base = (c * NS + s) * wins_per

        pltpu.async_copy(dst_hbm.at[pl.ds(base, K)], di_v.at[0], isem)

        @pl.loop(0, groups)
        def _(g):
            b = g & 1
            pltpu.make_async_copy(
                dst_hbm.at[pl.ds(base, K)], di_v.at[b], isem
            ).wait()

            for j in range(K):
                pltpu.async_copy(
                    val_v, acc.at[di_v.at[b].at[j]], ssems.at[b], add=True
                )

            @pl.when(g > 0)
            def _():
                for j in range(K):
                    pltpu.make_async_copy(
                        val_v, acc.at[di_v.at[1 - b].at[j]], ssems.at[1 - b]
                    ).wait()

            @pl.when(g + 1 < groups)
            def _():
                pltpu.async_copy(
                    dst_hbm.at[pl.ds(base + (g + 1) * K, K)],
                    di_v.at[1 - b], isem,
                )

        bl = (groups - 1) & 1
        for j in range(K):
            pltpu.make_async_copy(
                val_v, acc.at[di_v.at[bl].at[j]], ssems.at[bl]
            ).wait()
        plsc.subcore_barrier()
        pltpu.sync_copy(
            acc.at[pl.ds(s * stripe, stripe)],
            out_hbm.at[pl.ds(c * nt + s * stripe, stripe)],
        )

    return k(dst2d, ones_w, zeros_stripe)


def _edge_loop(src_hbm, dst_hbm, tab, acc, si_v, di_v, rows_v,
               isem, gsems, ssems, base, wins_per):
    groups = wins_per // K

    pltpu.async_copy(src_hbm.at[pl.ds(base, K)], si_v.at[0], isem)
    pltpu.async_copy(dst_hbm.at[pl.ds(base, K)], di_v.at[0], isem)

    @pl.loop(0, groups)
    def _(g):
        b = g & 1
        pltpu.make_async_copy(
            src_hbm.at[pl.ds(base, K)], si_v.at[b], isem
        ).wait()
        pltpu.make_async_copy(
            dst_hbm.at[pl.ds(base, K)], di_v.at[b], isem
        ).wait()

        for j in range(K):
            pltpu.async_copy(
                tab.at[si_v.at[b].at[j]], rows_v.at[b].at[j], gsems.at[j]
            )

        @pl.when(g > 0)
        def _():
            for j in range(K):
                pltpu.make_async_copy(
                    rows_v.at[1 - b].at[j],
                    acc.at[di_v.at[1 - b].at[j]],
                    ssems.at[1 - b],
                ).wait()

        @pl.when(g + 1 < groups)
        def _():
            pltpu.async_copy(
                src_hbm.at[pl.ds(base + (g + 1) * K, K)], si_v.at[1 - b], isem
            )
            pltpu.async_copy(
                dst_hbm.at[pl.ds(base + (g + 1) * K, K)], di_v.at[1 - b], isem
            )
        for j in range(K):
            pltpu.make_async_copy(
                tab.at[si_v.at[b].at[j]], rows_v.at[b].at[j], gsems.at[j]
            ).wait()
            pltpu.async_copy(
                rows_v.at[b].at[j], acc.at[di_v.at[b].at[j]],
                ssems.at[b], add=True,
            )

    bl = (groups - 1) & 1
    for j in range(K):
        pltpu.make_async_copy(
            rows_v.at[bl].at[j], acc.at[di_v.at[bl].at[j]], ssems.at[bl]
        ).wait()


def _sc_edge_half(src2d, dst2d, tab0, tab1, zeros_stripe, nt):
    wins_per = src2d.shape[0] // NS
    stripe = nt // NS

    @functools.partial(
        pl.kernel,
        out_type=jax.ShapeDtypeStruct((NC * nt, LN), jnp.float32),
        mesh=_mesh(),
        compiler_params=_SC_PARAMS,
        scratch_types=[
            pltpu.VMEM((2, K, WIN), jnp.int32),
            pltpu.VMEM((2, K, WIN), jnp.int32),
            pltpu.VMEM((2, K, WIN, LN), jnp.float32),
            pltpu.VMEM_SHARED((nt, LN), jnp.float32),
            pltpu.SemaphoreType.DMA,
            pltpu.SemaphoreType.DMA((K,)),
            pltpu.SemaphoreType.DMA((2,)),
        ],
    )
    def k(src_hbm, dst_hbm, t0_hbm, t1_hbm, zeros_hbm, out_hbm,
          si_v, di_v, rows_v, acc, isem, gsems, ssems):
        c = lax.axis_index("c")
        s = lax.axis_index("s")
        pltpu.sync_copy(zeros_hbm, acc.at[pl.ds(s * stripe, stripe)])
        plsc.subcore_barrier()
        base = s * wins_per

        @pl.when(c == 0)
        def _():
            _edge_loop(src_hbm, dst_hbm, t0_hbm, acc, si_v, di_v, rows_v,
                       isem, gsems, ssems, base, wins_per)

        @pl.when(c == 1)
        def _():
            _edge_loop(src_hbm, dst_hbm, t1_hbm, acc, si_v, di_v, rows_v,
                       isem, gsems, ssems, base, wins_per)

        plsc.subcore_barrier()
        pltpu.sync_copy(
            acc.at[pl.ds(s * stripe, stripe)],
            out_hbm.at[pl.ds(c * nt + s * stripe, stripe)],
        )

    return k(src2d, dst2d, tab0, tab1, zeros_stripe)


def _sc_edge_split(src2d, dst2d, tab, zeros_stripe, nt):
    wins_per = src2d.shape[0] // (NC * NS)
    stripe = nt // NS

    @functools.partial(
        pl.kernel,
        out_type=jax.ShapeDtypeStruct((NC * nt, LN), jnp.float32),
        mesh=_mesh(),
        compiler_params=_SC_PARAMS,
        scratch_types=[
            pltpu.VMEM((2, K, WIN), jnp.int32),
            pltpu.VMEM((2, K, WIN), jnp.int32),
            pltpu.VMEM((2, K, WIN, LN), jnp.float32),
            pltpu.VMEM_SHARED((nt, LN), jnp.float32),
            pltpu.SemaphoreType.DMA,
            pltpu.SemaphoreType.DMA((K,)),
            pltpu.SemaphoreType.DMA((2,)),
        ],
    )
    def k(src_hbm, dst_hbm, tab_hbm, zeros_hbm, out_hbm,
          si_v, di_v, rows_v, acc, isem, gsems, ssems):
        c = lax.axis_index("c")
        s = lax.axis_index("s")
        pltpu.sync_copy(zeros_hbm, acc.at[pl.ds(s * stripe, stripe)])
        plsc.subcore_barrier()
        base = (c * NS + s) * wins_per
        _edge_loop(src_hbm, dst_hbm, tab_hbm, acc, si_v, di_v, rows_v,
                   isem, gsems, ssems, base, wins_per)
        plsc.subcore_barrier()
        pltpu.sync_copy(
            acc.at[pl.ds(s * stripe, stripe)],
            out_hbm.at[pl.ds(c * nt + s * stripe, stripe)],
        )

    return k(src2d, dst2d, tab, zeros_stripe)


def _tc_mm(xpad, W1, nt):
    grid = nt // ROWB
    d_in, d_out = W1.shape

    def body(x_ref, w_ref, h_ref):
        h_ref[...] = jnp.dot(
            x_ref[...], w_ref[...], preferred_element_type=jnp.float32
        )

    return pl.pallas_call(
        body,
        grid=(grid,),
        in_specs=[
            pl.BlockSpec((ROWB, d_in), lambda i: (i, 0)),
            pl.BlockSpec((d_in, d_out), lambda i: (0, 0)),
        ],
        out_specs=pl.BlockSpec((ROWB, d_out), lambda i: (i, 0)),
        out_shape=jax.ShapeDtypeStruct((nt, d_out), jnp.float32),
    )(xpad, W1)


def _tc_prep(h1, degp, nt):
    grid = nt // ROWB
    d_out = h1.shape[1]

    def body(h_ref, dg_ref, dinv_ref, a_ref, b_ref):
        deg = dg_ref[0][:, 0:1] + dg_ref[1][:, 0:1] + 1.0
        dinv = lax.rsqrt(deg)
        hp = h_ref[...] * dinv
        dinv_ref[...] = dinv
        a_ref[...] = hp[:, : d_out // 2]
        b_ref[...] = hp[:, d_out // 2:]

    return pl.pallas_call(
        body,
        grid=(grid,),
        in_specs=[
            pl.BlockSpec((ROWB, d_out), lambda i: (i, 0)),
            pl.BlockSpec((2, ROWB, LN), lambda i: (0, i, 0)),
        ],
        out_specs=[
            pl.BlockSpec((ROWB, 1), lambda i: (i, 0)),
            pl.BlockSpec((ROWB, d_out // 2), lambda i: (i, 0)),
            pl.BlockSpec((ROWB, d_out // 2), lambda i: (i, 0)),
        ],
        out_shape=[
            jax.ShapeDtypeStruct((nt, 1), jnp.float32),
            jax.ShapeDtypeStruct((nt, d_out // 2), jnp.float32),
            jax.ShapeDtypeStruct((nt, d_out // 2), jnp.float32),
        ],
    )(h1, degp)


def _tc_mid(agg1, h1a, h1b, dinv, b1, W2, nt):
    grid = nt // ROWB
    d_hid, d_out = W2.shape

    def body(agg_ref, a_ref, b_ref, dinv_ref, b1_ref, w2_ref, out_ref):
        dv = dinv_ref[...]
        xa = (agg_ref[0] + a_ref[...]) * dv
        xb = (agg_ref[1] + b_ref[...]) * dv
        x2 = jnp.concatenate([xa, xb], axis=1) + b1_ref[...]
        x2 = jnp.maximum(x2, 0.0)
        h2 = jnp.dot(x2, w2_ref[...], preferred_element_type=jnp.float32)
        out_ref[...] = h2 * dv

    return pl.pallas_call(
        body,
        grid=(grid,),
        in_specs=[
            pl.BlockSpec((2, ROWB, LN), lambda i: (0, i, 0)),
            pl.BlockSpec((ROWB, LN), lambda i: (i, 0)),
            pl.BlockSpec((ROWB, LN), lambda i: (i, 0)),
            pl.BlockSpec((ROWB, 1), lambda i: (i, 0)),
            pl.BlockSpec((1, d_hid), lambda i: (0, 0)),
            pl.BlockSpec((d_hid, d_out), lambda i: (0, 0)),
        ],
        out_specs=pl.BlockSpec((ROWB, d_out), lambda i: (i, 0)),
        out_shape=jax.ShapeDtypeStruct((nt, d_out), jnp.float32),
    )(agg1, h1a, h1b, dinv, b1, W2)


def _tc_final(agg2, h2p, dinv, b2, nt):
    grid = nt // ROWB

    def body(agg_ref, h_ref, dinv_ref, b2_ref, out_ref):
        out_ref[...] = (
            (agg_ref[0] + agg_ref[1] + h_ref[...]) * dinv_ref[...] + b2_ref[...]
        )

    return pl.pallas_call(
        body,
        grid=(grid,),
        in_specs=[
            pl.BlockSpec((2, ROWB, LN), lambda i: (0, i, 0)),
            pl.BlockSpec((ROWB, LN), lambda i: (i, 0)),
            pl.BlockSpec((ROWB, 1), lambda i: (i, 0)),
            pl.BlockSpec((1, LN), lambda i: (0, 0)),
        ],
        out_specs=pl.BlockSpec((ROWB, LN), lambda i: (i, 0)),
        out_shape=jax.ShapeDtypeStruct((nt, LN), jnp.float32),
    )(agg2, h2p, dinv, b2)


def kernel(edge_index, user_emb, item_emb, W1, b1, W2, b2):
    nu = user_emb.shape[0]
    nn = nu + item_emb.shape[0]
    ne = edge_index.shape[1]
    nt = ((nn + ROWB) // ROWB) * ROWB
    chunk = NC * NS * WIN * K
    nep = -(-ne // chunk) * chunk

    src = jnp.pad(edge_index[0], (0, nep - ne), constant_values=nn)
    dst = jnp.pad(edge_index[1], (0, nep - ne), constant_values=nn)
    nw = nep // WIN
    src2d = src.reshape(nw // (NC * NS), NC * NS, WIN).transpose(1, 0, 2)
    src2d = src2d.reshape(nw, WIN)
    dst2d = dst.reshape(nw // (NC * NS), NC * NS, WIN).transpose(1, 0, 2)
    dst2d = dst2d.reshape(nw, WIN)

    x = jnp.concatenate([user_emb, item_emb], axis=0)
    xpad = jnp.pad(x, ((0, nt - nn), (0, 0)))
    zeros_stripe = jnp.zeros((nt // NS, LN), jnp.float32)
    ones_w = jnp.ones((WIN, LN), jnp.float32)

    h1 = _tc_mm(xpad, W1, nt)
    degp = _sc_hist(dst2d, ones_w, zeros_stripe, nt).reshape(NC, nt, LN)
    dinv, h1a, h1b = _tc_prep(h1, degp, nt)
    agg1 = _sc_edge_half(src2d, dst2d, h1a, h1b, zeros_stripe, nt).reshape(NC, nt, LN)
    h2p = _tc_mid(agg1, h1a, h1b, dinv, b1.reshape(1, -1), W2, nt)
    agg2 = _sc_edge_split(src2d, dst2d, h2p, zeros_stripe, nt).reshape(NC, nt, LN)
    out = _tc_final(agg2, h2p, dinv, b2.reshape(1, -1), nt)

    return out[:nu], out[nu:nn]
